# baseline (speedup 1.0000x reference)
import functools

import jax
import jax.numpy as jnp
from jax import lax
from jax.experimental import pallas as pl
from jax.experimental.pallas import tpu as pltpu
from jax.experimental.pallas import tpu_sc as plsc

EMBED_DIM = 64
NUM_WORKERS = 32
CHUNK = 512


def _build_gather(total_rows: int):
    b_per_w = total_rows // NUM_WORKERS
    n_chunks = b_per_w // CHUNK
    mesh = plsc.VectorSubcoreMesh(core_axis_name="c", subcore_axis_name="s")

    @functools.partial(
        pl.kernel,
        mesh=mesh,
        out_type=jax.ShapeDtypeStruct((total_rows, EMBED_DIM), jnp.float32),
        scratch_types=[
            pltpu.VMEM((CHUNK,), jnp.int32),
            pltpu.VMEM((CHUNK, EMBED_DIM), jnp.float32),
            pltpu.SemaphoreType.DMA,
        ],
        compiler_params=pltpu.CompilerParams(use_tc_tiling_on_sc=False),
    )
    def gather_kernel(idx_hbm, table_hbm, out_hbm, idx_v, rows_v, sem):
        wid = lax.axis_index("s") * 2 + lax.axis_index("c")
        base = wid * b_per_w

        def body(i, carry):
            off = base + i * CHUNK
            pltpu.sync_copy(idx_hbm.at[pl.ds(off, CHUNK)], idx_v)
            pltpu.async_copy(table_hbm.at[idx_v], rows_v, sem).wait()
            pltpu.sync_copy(rows_v, out_hbm.at[pl.ds(off, CHUNK)])
            return carry

        lax.fori_loop(0, n_chunks, body, 0)

    return gather_kernel


def kernel(seq, node2vec):
    batch, hist = seq.shape
    flat_idx = seq.reshape(-1).astype(jnp.int32)
    out = _build_gather(flat_idx.shape[0])(flat_idx, node2vec)
    return out.reshape(batch, hist, EMBED_DIM)


# speedup vs baseline: 1.0390x; 1.0390x over previous
import functools

import jax
import jax.numpy as jnp
from jax import lax
from jax.experimental import pallas as pl
from jax.experimental.pallas import tpu as pltpu
from jax.experimental.pallas import tpu_sc as plsc

EMBED_DIM = 64
NUM_WORKERS = 32
CHUNK = 640


def _build_gather(total_rows: int):
    b_per_w = total_rows // NUM_WORKERS
    n_chunks = b_per_w // CHUNK
    assert n_chunks % 2 == 0
    mesh = plsc.VectorSubcoreMesh(core_axis_name="c", subcore_axis_name="s")

    @functools.partial(
        pl.kernel,
        mesh=mesh,
        out_type=jax.ShapeDtypeStruct((total_rows, EMBED_DIM), jnp.float32),
        scratch_types=[
            pltpu.VMEM((b_per_w,), jnp.int32),
            pltpu.VMEM((CHUNK, EMBED_DIM), jnp.float32),
            pltpu.VMEM((CHUNK, EMBED_DIM), jnp.float32),
            pltpu.SemaphoreType.DMA,
            pltpu.SemaphoreType.DMA,
            pltpu.SemaphoreType.DMA,
            pltpu.SemaphoreType.DMA,
        ],
        compiler_params=pltpu.CompilerParams(use_tc_tiling_on_sc=False),
    )
    def gather_kernel(idx_hbm, table_hbm, out_hbm, idx_v, rows0, rows1,
                      sg0, sg1, so0, so1):
        wid = lax.axis_index("s") * 2 + lax.axis_index("c")
        base = wid * b_per_w
        pltpu.sync_copy(idx_hbm.at[pl.ds(base, b_per_w)], idx_v)

        def gather_desc(i, rows, sem):
            return pltpu.make_async_copy(
                table_hbm.at[idx_v.at[pl.ds(i * CHUNK, CHUNK)]], rows, sem)

        def out_desc(i, rows, sem):
            return pltpu.make_async_copy(
                rows, out_hbm.at[pl.ds(base + i * CHUNK, CHUNK)], sem)

        gather_desc(0, rows0, sg0).start()

        def body(g, carry):
            for b, rows, sg, so in ((0, rows0, sg0, so0), (1, rows1, sg1, so1)):
                i = 2 * g + b
                rows_o, sg_o, so_o = (rows1, sg1, so1) if b == 0 else (rows0, sg0, so0)
                gather_desc(i, rows, sg).wait()
                out_desc(i, rows, so).start()
                @pl.when(i >= 1)
                def _():
                    out_desc(i - 1, rows_o, so_o).wait()
                @pl.when(i < n_chunks - 1)
                def _():
                    gather_desc(i + 1, rows_o, sg_o).start()
            return carry

        lax.fori_loop(0, n_chunks // 2, body, 0)
        out_desc(n_chunks - 1, rows1, so1).wait()

    return gather_kernel


def kernel(seq, node2vec):
    batch, hist = seq.shape
    flat_idx = seq.reshape(-1).astype(jnp.int32)
    out = _build_gather(flat_idx.shape[0])(flat_idx, node2vec)
    return out.reshape(batch, hist, EMBED_DIM)
